# Initial kernel scaffold; baseline (speedup 1.0000x reference)
import functools
import jax
import jax.numpy as jnp
from jax import lax
from jax.experimental import pallas as pl
from jax.experimental.pallas import tpu as pltpu

N = 50000
E = 800000
L = 10000
P = 40000
NODE_DIM = 128
HID = 64
EDGE_DIM = 4
C_LIG = 2000
C_PRO = 8000

N_PAD = 50176
BLK_N = 1568
E_PAD = 802816
BLK_E = 4096
TW = 80


def _silu(v):
    return v * jax.nn.sigmoid(v)


def _dot(a, b):
    return jnp.dot(a, b, preferred_element_type=jnp.float32)


def _mlp_in_body(x_ref, w1_ref, b1_ref, w2_ref, b2_ref, o_ref):
    h = _silu(_dot(x_ref[...], w1_ref[...]) + b1_ref[...])
    o_ref[...] = _dot(h, w2_ref[...]) + b2_ref[...]


def _mlp_in(x, w1, b1, w2, b2):
    grid = (N_PAD // BLK_N,)
    return pl.pallas_call(
        _mlp_in_body,
        grid=grid,
        in_specs=[
            pl.BlockSpec((BLK_N, NODE_DIM), lambda i: (i, 0)),
            pl.BlockSpec((NODE_DIM, HID), lambda i: (0, 0)),
            pl.BlockSpec((1, HID), lambda i: (0, 0)),
            pl.BlockSpec((HID, HID), lambda i: (0, 0)),
            pl.BlockSpec((1, HID), lambda i: (0, 0)),
        ],
        out_specs=pl.BlockSpec((BLK_N, HID), lambda i: (i, 0)),
        out_shape=jax.ShapeDtypeStruct((N_PAD, HID), jnp.float32),
    )(x, w1, b1.reshape(1, HID), w2, b2.reshape(1, HID))


def _prep_body(h_ref, c_ref, wr_ref, wc_ref, t1_ref, t2_ref):
    h = h_ref[...]
    c = c_ref[...]
    t1_ref[...] = jnp.concatenate([_dot(h, wr_ref[...]), c], axis=1)
    t2_ref[...] = jnp.concatenate([_dot(h, wc_ref[...]), c], axis=1)


def _prep(h, coordp, wr, wc):
    grid = (N_PAD // BLK_N,)
    return pl.pallas_call(
        _prep_body,
        grid=grid,
        in_specs=[
            pl.BlockSpec((BLK_N, HID), lambda i: (i, 0)),
            pl.BlockSpec((BLK_N, 16), lambda i: (i, 0)),
            pl.BlockSpec((HID, HID), lambda i: (0, 0)),
            pl.BlockSpec((HID, HID), lambda i: (0, 0)),
        ],
        out_specs=[
            pl.BlockSpec((BLK_N, TW), lambda i: (i, 0)),
            pl.BlockSpec((BLK_N, TW), lambda i: (i, 0)),
        ],
        out_shape=[
            jax.ShapeDtypeStruct((N_PAD, TW), jnp.float32),
            jax.ShapeDtypeStruct((N_PAD, TW), jnp.float32),
        ],
    )(h, coordp, wr, wc)


def _edge_body(a_ref, b_ref, ea_ref, wrad_ref, wea_ref, be1_ref, we2_ref,
               be2_ref, wc1_ref, bc1_ref, wc2_ref, o_ref):
    a = a_ref[...]
    b = b_ref[...]
    af = a[:, :HID]
    bf = b[:, :HID]
    ac = a[:, HID:TW]
    bc = b[:, HID:TW]
    cdiff = ac - bc
    radial = jnp.sum(cdiff * cdiff, axis=1, keepdims=True)
    z = af + bf + radial * wrad_ref[...] + _dot(ea_ref[...], wea_ref[...]) \
        + be1_ref[...]
    m1 = _silu(z)
    m = _silu(_dot(m1, we2_ref[...]) + be2_ref[...])
    cm = _silu(_dot(m, wc1_ref[...]) + bc1_ref[...])
    s = _dot(cm, wc2_ref[...])
    lane = lax.broadcasted_iota(jnp.int32, (1, 16), 1)
    one3 = jnp.where(lane == 3, 1.0, 0.0).astype(jnp.float32)
    tr = cdiff * s + one3
    o_ref[...] = jnp.concatenate([m, tr], axis=1)


def _edge(A, B, EA, p):
    grid = (E_PAD // BLK_E,)
    wrad = p['We1'][2 * HID].reshape(1, HID)
    wea = p['We1'][2 * HID + 1:]
    return pl.pallas_call(
        _edge_body,
        grid=grid,
        in_specs=[
            pl.BlockSpec((BLK_E, TW), lambda i: (i, 0)),
            pl.BlockSpec((BLK_E, TW), lambda i: (i, 0)),
            pl.BlockSpec((BLK_E, EDGE_DIM), lambda i: (i, 0)),
            pl.BlockSpec((1, HID), lambda i: (0, 0)),
            pl.BlockSpec((EDGE_DIM, HID), lambda i: (0, 0)),
            pl.BlockSpec((1, HID), lambda i: (0, 0)),
            pl.BlockSpec((HID, HID), lambda i: (0, 0)),
            pl.BlockSpec((1, HID), lambda i: (0, 0)),
            pl.BlockSpec((HID, HID), lambda i: (0, 0)),
            pl.BlockSpec((1, HID), lambda i: (0, 0)),
            pl.BlockSpec((HID, 1), lambda i: (0, 0)),
        ],
        out_specs=pl.BlockSpec((BLK_E, TW), lambda i: (i, 0)),
        out_shape=jax.ShapeDtypeStruct((E_PAD, TW), jnp.float32),
    )(A, B, EA, wrad, wea, p['be1'].reshape(1, HID), p['We2'],
      p['be2'].reshape(1, HID), p['Wc1'], p['bc1'].reshape(1, HID), p['Wc2'])


def _node_body(h_ref, c_ref, agg_ref, wa_ref, wb_ref, bn1_ref, wn2_ref,
               bn2_ref, ho_ref, co_ref):
    h = h_ref[...]
    agg = agg_ref[...]
    am = agg[:, :HID]
    at = agg[:, HID:TW]
    cnt = at[:, 3:4]
    recip = 1.0 / jnp.maximum(cnt, 1.0)
    lane = lax.broadcasted_iota(jnp.int32, (1, 16), 1)
    mask3 = jnp.where(lane < 3, 1.0, 0.0).astype(jnp.float32)
    co_ref[...] = c_ref[...] + at * recip * mask3
    t = _silu(_dot(h, wa_ref[...]) + _dot(am, wb_ref[...]) + bn1_ref[...])
    ho_ref[...] = h + _dot(t, wn2_ref[...]) + bn2_ref[...]


def _node(h, coordp, AGG, p):
    grid = (N_PAD // BLK_N,)
    return pl.pallas_call(
        _node_body,
        grid=grid,
        in_specs=[
            pl.BlockSpec((BLK_N, HID), lambda i: (i, 0)),
            pl.BlockSpec((BLK_N, 16), lambda i: (i, 0)),
            pl.BlockSpec((BLK_N, TW), lambda i: (i, 0)),
            pl.BlockSpec((HID, HID), lambda i: (0, 0)),
            pl.BlockSpec((HID, HID), lambda i: (0, 0)),
            pl.BlockSpec((1, HID), lambda i: (0, 0)),
            pl.BlockSpec((HID, HID), lambda i: (0, 0)),
            pl.BlockSpec((1, HID), lambda i: (0, 0)),
        ],
        out_specs=[
            pl.BlockSpec((BLK_N, HID), lambda i: (i, 0)),
            pl.BlockSpec((BLK_N, 16), lambda i: (i, 0)),
        ],
        out_shape=[
            jax.ShapeDtypeStruct((N_PAD, HID), jnp.float32),
            jax.ShapeDtypeStruct((N_PAD, 16), jnp.float32),
        ],
    )(h, coordp, AGG, p['Wn1'][:HID], p['Wn1'][HID:],
      p['bn1'].reshape(1, HID), p['Wn2'], p['bn2'].reshape(1, HID))


def _post_body(h_ref, w_ref, b_ref, o_ref):
    o_ref[...] = _dot(h_ref[...], w_ref[...]) + b_ref[...]


def _post(h, w, b):
    grid = (N_PAD // BLK_N,)
    return pl.pallas_call(
        _post_body,
        grid=grid,
        in_specs=[
            pl.BlockSpec((BLK_N, HID), lambda i: (i, 0)),
            pl.BlockSpec((HID, HID), lambda i: (0, 0)),
            pl.BlockSpec((1, HID), lambda i: (0, 0)),
        ],
        out_specs=pl.BlockSpec((BLK_N, HID), lambda i: (i, 0)),
        out_shape=jax.ShapeDtypeStruct((N_PAD, HID), jnp.float32),
    )(h, w, b.reshape(1, HID))


def _gat_prep_body(x_ref, wl_ref, bl_ref, br_ref, att_ref, xl_ref, e_ref,
                   mx_ref):
    i = pl.program_id(0)
    xl = _dot(x_ref[...], wl_ref[...]) + bl_ref[...]
    xl_ref[...] = xl
    e = _dot(jax.nn.leaky_relu(xl + br_ref[...], 0.2), att_ref[...])
    e_ref[...] = e

    @pl.when(i == 0)
    def _init():
        mx_ref[...] = jnp.full_like(mx_ref[...], -jnp.inf)

    mx_ref[0, 0] = jnp.maximum(mx_ref[0, 0], jnp.max(e))


def _gat_prep(x, gp, blk):
    m = x.shape[0]
    grid = (m // blk,)
    return pl.pallas_call(
        _gat_prep_body,
        grid=grid,
        in_specs=[
            pl.BlockSpec((blk, HID), lambda i: (i, 0)),
            pl.BlockSpec((HID, HID), lambda i: (0, 0)),
            pl.BlockSpec((1, HID), lambda i: (0, 0)),
            pl.BlockSpec((1, HID), lambda i: (0, 0)),
            pl.BlockSpec((HID, 1), lambda i: (0, 0)),
        ],
        out_specs=[
            pl.BlockSpec((blk, HID), lambda i: (i, 0)),
            pl.BlockSpec((blk, 1), lambda i: (i, 0)),
            pl.BlockSpec((1, 1), lambda i: (0, 0)),
        ],
        out_shape=[
            jax.ShapeDtypeStruct((m, HID), jnp.float32),
            jax.ShapeDtypeStruct((m, 1), jnp.float32),
            jax.ShapeDtypeStruct((1, 1), jnp.float32),
        ],
    )(x, gp['Wl'], gp['bl'].reshape(1, HID), gp['br'].reshape(1, HID),
      gp['att'].reshape(HID, 1))


def _gat_w_body(xl_ref, e_ref, mx_ref, o_ref):
    w = jnp.exp(e_ref[...] - mx_ref[0, 0])
    lane = lax.broadcasted_iota(jnp.int32, (1, 16), 1)
    one0 = jnp.where(lane == 0, 1.0, 0.0).astype(jnp.float32)
    o_ref[...] = jnp.concatenate([w * xl_ref[...], w * one0], axis=1)


def _gat_w(xl, e, mx, blk):
    m = xl.shape[0]
    grid = (m // blk,)
    return pl.pallas_call(
        _gat_w_body,
        grid=grid,
        in_specs=[
            pl.BlockSpec((blk, HID), lambda i: (i, 0)),
            pl.BlockSpec((blk, 1), lambda i: (i, 0)),
            pl.BlockSpec((1, 1), lambda i: (0, 0)),
        ],
        out_specs=pl.BlockSpec((blk, TW), lambda i: (i, 0)),
        out_shape=jax.ShapeDtypeStruct((m, TW), jnp.float32),
    )(xl, e, mx)


def _gat_out_body(agg_ref, mx_ref, c0_ref, bl_ref, bias_ref, o_ref):
    wself = jnp.exp(c0_ref[0, 0] - mx_ref[0, 0])
    numer = agg_ref[:, :HID] + wself * bl_ref[...]
    denom = agg_ref[:, HID:HID + 1] + wself + 1e-16
    o_ref[...] = numer / denom + bias_ref[...]


def _gat_out(agg, mx, c0, gp, blk):
    c = agg.shape[0]
    grid = (c // blk,)
    return pl.pallas_call(
        _gat_out_body,
        grid=grid,
        in_specs=[
            pl.BlockSpec((blk, TW), lambda i: (i, 0)),
            pl.BlockSpec((1, 1), lambda i: (0, 0)),
            pl.BlockSpec((1, 1), lambda i: (0, 0)),
            pl.BlockSpec((1, HID), lambda i: (0, 0)),
            pl.BlockSpec((1, HID), lambda i: (0, 0)),
        ],
        out_specs=pl.BlockSpec((blk, HID), lambda i: (i, 0)),
        out_shape=jax.ShapeDtypeStruct((c, HID), jnp.float32),
    )(agg, mx, c0, gp['bl'].reshape(1, HID), gp['bias'].reshape(1, HID))


def _gather_rows(table, idx):
    return jnp.take(table, idx, axis=0)


def _scatter_add(values, ids, num_segments):
    return jax.ops.segment_sum(values, ids, num_segments=num_segments)


@jax.jit
def _run(x, pos, edge_index, edge_attr, batch_lig, batch_pro,
         lin_W, lin_b, W_in, b_in, layers, W_out, b_out, gat_lig, gat_pro):
    f32 = jnp.float32
    xp = jnp.zeros((N_PAD, NODE_DIM), f32).at[:N].set(x)
    coordp = jnp.zeros((N_PAD, 16), f32).at[:N, :3].set(pos)
    row = edge_index[0]
    col = edge_index[1]
    rowp = jnp.full((E_PAD,), N, jnp.int32).at[:E].set(row)
    colp = jnp.full((E_PAD,), N, jnp.int32).at[:E].set(col)
    eap = jnp.zeros((E_PAD, EDGE_DIM), f32).at[:E].set(edge_attr)

    h = _mlp_in(xp, lin_W, lin_b, W_in, b_in)
    for p in layers:
        T1, T2 = _prep(h, coordp, p['We1'][:HID], p['We1'][HID:2 * HID])
        A = _gather_rows(T1, rowp)
        B = _gather_rows(T2, colp)
        S = _edge(A, B, eap, p)
        AGG = _scatter_add(S, rowp, N_PAD)
        h, coordp = _node(h, coordp, AGG, p)

    hout = _post(h, W_out, b_out)

    def gat(xa, batch, c, blk_a, blk_c, gp):
        xl, e, mx = _gat_prep(xa, gp, blk_a)
        c0 = jnp.dot(jax.nn.leaky_relu(gp['bl'] + gp['br'], 0.2),
                     gp['att']).reshape(1, 1)
        mx = jnp.maximum(mx, c0)
        S = _gat_w(xl, e, mx, blk_a)
        agg = _scatter_add(S, batch, c)
        return _gat_out(agg, mx, c0, gp, blk_c)

    x_lig_cg = gat(hout[:L], batch_lig, C_LIG, 1250, 1000, gat_lig)
    x_pro_cg = gat(hout[L:L + P], batch_pro, C_PRO, 1600, 1000, gat_pro)
    pos_lig = coordp[:L, :3]
    pos_pro = coordp[L:L + P, :3]
    return x_lig_cg, x_pro_cg, pos_lig, pos_pro


def kernel(x, pos, edge_index, edge_attr, batch_lig, batch_pro,
           lin_W, lin_b, W_in, b_in, layers, W_out, b_out, gat_lig, gat_pro):
    return _run(x, pos, edge_index, edge_attr, batch_lig, batch_pro,
                lin_W, lin_b, W_in, b_in, layers, W_out, b_out,
                gat_lig, gat_pro)


# speedup vs baseline: 1.2424x; 1.2424x over previous
import functools
import jax
import jax.numpy as jnp
from jax import lax
from jax.experimental import pallas as pl
from jax.experimental.pallas import tpu as pltpu

N = 50000
E = 800000
L = 10000
P = 40000
NODE_DIM = 128
HID = 64
EDGE_DIM = 4
C_LIG = 2000
C_PRO = 8000

N_PAD = 50176
BLK_N = 1568
E_PAD = 802816
BLK_E = 4096
TW = 80


def _silu(v):
    return v * jax.nn.sigmoid(v)


def _dot(a, b):
    return jnp.dot(a, b, preferred_element_type=jnp.float32)


def _mlp_in_body(x_ref, w1_ref, b1_ref, w2_ref, b2_ref, o_ref):
    h = _silu(_dot(x_ref[...], w1_ref[...]) + b1_ref[...])
    o_ref[...] = _dot(h, w2_ref[...]) + b2_ref[...]


def _mlp_in(x, w1, b1, w2, b2):
    grid = (N_PAD // BLK_N,)
    return pl.pallas_call(
        _mlp_in_body,
        grid=grid,
        in_specs=[
            pl.BlockSpec((BLK_N, NODE_DIM), lambda i: (i, 0)),
            pl.BlockSpec((NODE_DIM, HID), lambda i: (0, 0)),
            pl.BlockSpec((1, HID), lambda i: (0, 0)),
            pl.BlockSpec((HID, HID), lambda i: (0, 0)),
            pl.BlockSpec((1, HID), lambda i: (0, 0)),
        ],
        out_specs=pl.BlockSpec((BLK_N, HID), lambda i: (i, 0)),
        out_shape=jax.ShapeDtypeStruct((N_PAD, HID), jnp.float32),
    )(x, w1, b1.reshape(1, HID), w2, b2.reshape(1, HID))


def _prep_body(h_ref, c_ref, wr_ref, wc_ref, t1_ref, t2_ref):
    h = h_ref[...]
    c = c_ref[...]
    t1_ref[...] = jnp.concatenate([_dot(h, wr_ref[...]), c], axis=1)
    t2_ref[...] = jnp.concatenate([_dot(h, wc_ref[...]), c], axis=1)


def _prep(h, coordp, wr, wc):
    grid = (N_PAD // BLK_N,)
    return pl.pallas_call(
        _prep_body,
        grid=grid,
        in_specs=[
            pl.BlockSpec((BLK_N, HID), lambda i: (i, 0)),
            pl.BlockSpec((BLK_N, 16), lambda i: (i, 0)),
            pl.BlockSpec((HID, HID), lambda i: (0, 0)),
            pl.BlockSpec((HID, HID), lambda i: (0, 0)),
        ],
        out_specs=[
            pl.BlockSpec((BLK_N, TW), lambda i: (i, 0)),
            pl.BlockSpec((BLK_N, TW), lambda i: (i, 0)),
        ],
        out_shape=[
            jax.ShapeDtypeStruct((N_PAD, TW), jnp.float32),
            jax.ShapeDtypeStruct((N_PAD, TW), jnp.float32),
        ],
    )(h, coordp, wr, wc)


def _edge_body(a_ref, b_ref, ea_ref, wrad_ref, wea_ref, be1_ref, we2_ref,
               be2_ref, wc1_ref, bc1_ref, wc2_ref, o_ref):
    a = a_ref[...]
    b = b_ref[...]
    af = a[:, :HID]
    bf = b[:, :HID]
    ac = a[:, HID:TW]
    bc = b[:, HID:TW]
    cdiff = ac - bc
    radial = jnp.sum(cdiff * cdiff, axis=1, keepdims=True)
    z = af + bf + radial * wrad_ref[...] + _dot(ea_ref[...], wea_ref[...]) \
        + be1_ref[...]
    m1 = _silu(z)
    m = _silu(_dot(m1, we2_ref[...]) + be2_ref[...])
    cm = _silu(_dot(m, wc1_ref[...]) + bc1_ref[...])
    s = _dot(cm, wc2_ref[...])
    lane = lax.broadcasted_iota(jnp.int32, (1, 16), 1)
    one3 = jnp.where(lane == 3, 1.0, 0.0).astype(jnp.float32)
    tr = cdiff * s + one3
    o_ref[...] = jnp.concatenate([m, tr], axis=1)


def _edge(A, B, EA, p):
    grid = (E_PAD // BLK_E,)
    wrad = p['We1'][2 * HID].reshape(1, HID)
    wea = p['We1'][2 * HID + 1:]
    return pl.pallas_call(
        _edge_body,
        grid=grid,
        in_specs=[
            pl.BlockSpec((BLK_E, TW), lambda i: (i, 0)),
            pl.BlockSpec((BLK_E, TW), lambda i: (i, 0)),
            pl.BlockSpec((BLK_E, EDGE_DIM), lambda i: (i, 0)),
            pl.BlockSpec((1, HID), lambda i: (0, 0)),
            pl.BlockSpec((EDGE_DIM, HID), lambda i: (0, 0)),
            pl.BlockSpec((1, HID), lambda i: (0, 0)),
            pl.BlockSpec((HID, HID), lambda i: (0, 0)),
            pl.BlockSpec((1, HID), lambda i: (0, 0)),
            pl.BlockSpec((HID, HID), lambda i: (0, 0)),
            pl.BlockSpec((1, HID), lambda i: (0, 0)),
            pl.BlockSpec((HID, 1), lambda i: (0, 0)),
        ],
        out_specs=pl.BlockSpec((BLK_E, TW), lambda i: (i, 0)),
        out_shape=jax.ShapeDtypeStruct((E_PAD, TW), jnp.float32),
    )(A, B, EA, wrad, wea, p['be1'].reshape(1, HID), p['We2'],
      p['be2'].reshape(1, HID), p['Wc1'], p['bc1'].reshape(1, HID), p['Wc2'])


def _node_body(h_ref, c_ref, agg_ref, wa_ref, wb_ref, bn1_ref, wn2_ref,
               bn2_ref, ho_ref, co_ref):
    h = h_ref[...]
    agg = agg_ref[...]
    am = agg[:, :HID]
    at = agg[:, HID:TW]
    cnt = at[:, 3:4]
    recip = 1.0 / jnp.maximum(cnt, 1.0)
    lane = lax.broadcasted_iota(jnp.int32, (1, 16), 1)
    mask3 = jnp.where(lane < 3, 1.0, 0.0).astype(jnp.float32)
    co_ref[...] = c_ref[...] + at * recip * mask3
    t = _silu(_dot(h, wa_ref[...]) + _dot(am, wb_ref[...]) + bn1_ref[...])
    ho_ref[...] = h + _dot(t, wn2_ref[...]) + bn2_ref[...]


def _node(h, coordp, AGG, p):
    grid = (N_PAD // BLK_N,)
    return pl.pallas_call(
        _node_body,
        grid=grid,
        in_specs=[
            pl.BlockSpec((BLK_N, HID), lambda i: (i, 0)),
            pl.BlockSpec((BLK_N, 16), lambda i: (i, 0)),
            pl.BlockSpec((BLK_N, TW), lambda i: (i, 0)),
            pl.BlockSpec((HID, HID), lambda i: (0, 0)),
            pl.BlockSpec((HID, HID), lambda i: (0, 0)),
            pl.BlockSpec((1, HID), lambda i: (0, 0)),
            pl.BlockSpec((HID, HID), lambda i: (0, 0)),
            pl.BlockSpec((1, HID), lambda i: (0, 0)),
        ],
        out_specs=[
            pl.BlockSpec((BLK_N, HID), lambda i: (i, 0)),
            pl.BlockSpec((BLK_N, 16), lambda i: (i, 0)),
        ],
        out_shape=[
            jax.ShapeDtypeStruct((N_PAD, HID), jnp.float32),
            jax.ShapeDtypeStruct((N_PAD, 16), jnp.float32),
        ],
    )(h, coordp, AGG, p['Wn1'][:HID], p['Wn1'][HID:],
      p['bn1'].reshape(1, HID), p['Wn2'], p['bn2'].reshape(1, HID))


def _post_body(h_ref, w_ref, b_ref, o_ref):
    o_ref[...] = _dot(h_ref[...], w_ref[...]) + b_ref[...]


def _post(h, w, b):
    grid = (N_PAD // BLK_N,)
    return pl.pallas_call(
        _post_body,
        grid=grid,
        in_specs=[
            pl.BlockSpec((BLK_N, HID), lambda i: (i, 0)),
            pl.BlockSpec((HID, HID), lambda i: (0, 0)),
            pl.BlockSpec((1, HID), lambda i: (0, 0)),
        ],
        out_specs=pl.BlockSpec((BLK_N, HID), lambda i: (i, 0)),
        out_shape=jax.ShapeDtypeStruct((N_PAD, HID), jnp.float32),
    )(h, w, b.reshape(1, HID))


def _gat_prep_body(x_ref, wl_ref, bl_ref, br_ref, att_ref, xl_ref, e_ref,
                   mx_ref):
    i = pl.program_id(0)
    xl = _dot(x_ref[...], wl_ref[...]) + bl_ref[...]
    xl_ref[...] = xl
    e = _dot(jax.nn.leaky_relu(xl + br_ref[...], 0.2), att_ref[...])
    e_ref[...] = e

    @pl.when(i == 0)
    def _init():
        mx_ref[...] = jnp.full_like(mx_ref[...], -jnp.inf)

    mx_ref[...] = jnp.maximum(mx_ref[...], jnp.max(e))


def _gat_prep(x, gp, blk):
    m = x.shape[0]
    grid = (m // blk,)
    return pl.pallas_call(
        _gat_prep_body,
        grid=grid,
        in_specs=[
            pl.BlockSpec((blk, HID), lambda i: (i, 0)),
            pl.BlockSpec((HID, HID), lambda i: (0, 0)),
            pl.BlockSpec((1, HID), lambda i: (0, 0)),
            pl.BlockSpec((1, HID), lambda i: (0, 0)),
            pl.BlockSpec((HID, 1), lambda i: (0, 0)),
        ],
        out_specs=[
            pl.BlockSpec((blk, HID), lambda i: (i, 0)),
            pl.BlockSpec((blk, 1), lambda i: (i, 0)),
            pl.BlockSpec((1, 1), lambda i: (0, 0)),
        ],
        out_shape=[
            jax.ShapeDtypeStruct((m, HID), jnp.float32),
            jax.ShapeDtypeStruct((m, 1), jnp.float32),
            jax.ShapeDtypeStruct((1, 1), jnp.float32),
        ],
    )(x, gp['Wl'], gp['bl'].reshape(1, HID), gp['br'].reshape(1, HID),
      gp['att'].reshape(HID, 1))


def _gat_w_body(xl_ref, e_ref, mx_ref, o_ref):
    w = jnp.exp(e_ref[...] - mx_ref[...])
    lane = lax.broadcasted_iota(jnp.int32, (1, 16), 1)
    one0 = jnp.where(lane == 0, 1.0, 0.0).astype(jnp.float32)
    o_ref[...] = jnp.concatenate([w * xl_ref[...], w * one0], axis=1)


def _gat_w(xl, e, mx, blk):
    m = xl.shape[0]
    grid = (m // blk,)
    return pl.pallas_call(
        _gat_w_body,
        grid=grid,
        in_specs=[
            pl.BlockSpec((blk, HID), lambda i: (i, 0)),
            pl.BlockSpec((blk, 1), lambda i: (i, 0)),
            pl.BlockSpec((1, 1), lambda i: (0, 0)),
        ],
        out_specs=pl.BlockSpec((blk, TW), lambda i: (i, 0)),
        out_shape=jax.ShapeDtypeStruct((m, TW), jnp.float32),
    )(xl, e, mx)


def _gat_out_body(agg_ref, mx_ref, c0_ref, bl_ref, bias_ref, o_ref):
    wself = jnp.exp(c0_ref[...] - mx_ref[...])
    numer = agg_ref[:, :HID] + wself * bl_ref[...]
    denom = agg_ref[:, HID:HID + 1] + wself + 1e-16
    o_ref[...] = numer / denom + bias_ref[...]


def _gat_out(agg, mx, c0, gp, blk):
    c = agg.shape[0]
    grid = (c // blk,)
    return pl.pallas_call(
        _gat_out_body,
        grid=grid,
        in_specs=[
            pl.BlockSpec((blk, TW), lambda i: (i, 0)),
            pl.BlockSpec((1, 1), lambda i: (0, 0)),
            pl.BlockSpec((1, 1), lambda i: (0, 0)),
            pl.BlockSpec((1, HID), lambda i: (0, 0)),
            pl.BlockSpec((1, HID), lambda i: (0, 0)),
        ],
        out_specs=pl.BlockSpec((blk, HID), lambda i: (i, 0)),
        out_shape=jax.ShapeDtypeStruct((c, HID), jnp.float32),
    )(agg, mx, c0, gp['bl'].reshape(1, HID), gp['bias'].reshape(1, HID))


def _gather_rows(table, idx):
    return jnp.take(table, idx, axis=0)


def _scatter_add(values, ids, num_segments):
    return jax.ops.segment_sum(values, ids, num_segments=num_segments)


@jax.jit
def _run(x, pos, edge_index, edge_attr, batch_lig, batch_pro,
         lin_W, lin_b, W_in, b_in, layers, W_out, b_out, gat_lig, gat_pro):
    f32 = jnp.float32
    xp = jnp.zeros((N_PAD, NODE_DIM), f32).at[:N].set(x)
    coordp = jnp.zeros((N_PAD, 16), f32).at[:N, :3].set(pos)
    row = edge_index[0]
    col = edge_index[1]
    rowp = jnp.full((E_PAD,), N, jnp.int32).at[:E].set(row)
    colp = jnp.full((E_PAD,), N, jnp.int32).at[:E].set(col)
    eap = jnp.zeros((E_PAD, EDGE_DIM), f32).at[:E].set(edge_attr)

    h = _mlp_in(xp, lin_W, lin_b, W_in, b_in)
    for p in layers:
        T1, T2 = _prep(h, coordp, p['We1'][:HID], p['We1'][HID:2 * HID])
        A = _gather_rows(T1, rowp)
        B = _gather_rows(T2, colp)
        S = _edge(A, B, eap, p)
        AGG = _scatter_add(S, rowp, N_PAD)
        h, coordp = _node(h, coordp, AGG, p)

    hout = _post(h, W_out, b_out)

    def gat(xa, batch, c, blk_a, blk_c, gp):
        xl, e, mx = _gat_prep(xa, gp, blk_a)
        c0 = jnp.dot(jax.nn.leaky_relu(gp['bl'] + gp['br'], 0.2),
                     gp['att']).reshape(1, 1)
        mx = jnp.maximum(mx, c0)
        S = _gat_w(xl, e, mx, blk_a)
        agg = _scatter_add(S, batch, c)
        return _gat_out(agg, mx, c0, gp, blk_c)

    x_lig_cg = gat(hout[:L], batch_lig, C_LIG, 1000, 1000, gat_lig)
    x_pro_cg = gat(hout[L:L + P], batch_pro, C_PRO, 1600, 1000, gat_pro)
    pos_lig = coordp[:L, :3]
    pos_pro = coordp[L:L + P, :3]
    return x_lig_cg, x_pro_cg, pos_lig, pos_pro


def kernel(x, pos, edge_index, edge_attr, batch_lig, batch_pro,
           lin_W, lin_b, W_in, b_in, layers, W_out, b_out, gat_lig, gat_pro):
    return _run(x, pos, edge_index, edge_attr, batch_lig, batch_pro,
                lin_W, lin_b, W_in, b_in, layers, W_out, b_out,
                gat_lig, gat_pro)


# speedup vs baseline: 1.9586x; 1.5765x over previous
import functools
import jax
import jax.numpy as jnp
from jax import lax
from jax.experimental import pallas as pl
from jax.experimental.pallas import tpu as pltpu
from jax.experimental.pallas import tpu_sc as plsc

N = 50000
E = 800000
L = 10000
P = 40000
NODE_DIM = 128
HID = 64
EDGE_DIM = 4
C_LIG = 2000
C_PRO = 8000

N_PAD = 50176
BLK_N = 1568
E_PAD = 819200
BLK_E = 4096
TW = 80
GW = 128


def _silu(v):
    return v * jax.nn.sigmoid(v)


def _dot(a, b):
    return jnp.dot(a, b, preferred_element_type=jnp.float32)


def _mlp_in_body(x_ref, w1_ref, b1_ref, w2_ref, b2_ref, o_ref):
    h = _silu(_dot(x_ref[...], w1_ref[...]) + b1_ref[...])
    o_ref[...] = _dot(h, w2_ref[...]) + b2_ref[...]


def _mlp_in(x, w1, b1, w2, b2):
    grid = (N_PAD // BLK_N,)
    return pl.pallas_call(
        _mlp_in_body,
        grid=grid,
        in_specs=[
            pl.BlockSpec((BLK_N, NODE_DIM), lambda i: (i, 0)),
            pl.BlockSpec((NODE_DIM, HID), lambda i: (0, 0)),
            pl.BlockSpec((1, HID), lambda i: (0, 0)),
            pl.BlockSpec((HID, HID), lambda i: (0, 0)),
            pl.BlockSpec((1, HID), lambda i: (0, 0)),
        ],
        out_specs=pl.BlockSpec((BLK_N, HID), lambda i: (i, 0)),
        out_shape=jax.ShapeDtypeStruct((N_PAD, HID), jnp.float32),
    )(x, w1, b1.reshape(1, HID), w2, b2.reshape(1, HID))


def _prep_body(h_ref, c_ref, wr_ref, wc_ref, t1_ref, t2_ref):
    h = h_ref[...]
    c = c_ref[...]
    z = jnp.zeros((h.shape[0], GW - TW), jnp.float32)
    t1_ref[...] = jnp.concatenate([_dot(h, wr_ref[...]), c, z], axis=1)
    t2_ref[...] = jnp.concatenate([_dot(h, wc_ref[...]), c, z], axis=1)


def _prep(h, coordp, wr, wc):
    grid = (N_PAD // BLK_N,)
    return pl.pallas_call(
        _prep_body,
        grid=grid,
        in_specs=[
            pl.BlockSpec((BLK_N, HID), lambda i: (i, 0)),
            pl.BlockSpec((BLK_N, 16), lambda i: (i, 0)),
            pl.BlockSpec((HID, HID), lambda i: (0, 0)),
            pl.BlockSpec((HID, HID), lambda i: (0, 0)),
        ],
        out_specs=[
            pl.BlockSpec((BLK_N, GW), lambda i: (i, 0)),
            pl.BlockSpec((BLK_N, GW), lambda i: (i, 0)),
        ],
        out_shape=[
            jax.ShapeDtypeStruct((N_PAD, GW), jnp.float32),
            jax.ShapeDtypeStruct((N_PAD, GW), jnp.float32),
        ],
    )(h, coordp, wr, wc)


def _edge_body(a_ref, b_ref, ea_ref, wrad_ref, wea_ref, be1_ref, we2_ref,
               be2_ref, wc1_ref, bc1_ref, wc2_ref, o_ref):
    a = a_ref[...]
    b = b_ref[...]
    af = a[:, :HID]
    bf = b[:, :HID]
    ac = a[:, HID:TW]
    bc = b[:, HID:TW]
    cdiff = ac - bc
    radial = jnp.sum(cdiff * cdiff, axis=1, keepdims=True)
    z = af + bf + radial * wrad_ref[...] + _dot(ea_ref[...], wea_ref[...]) \
        + be1_ref[...]
    m1 = _silu(z)
    m = _silu(_dot(m1, we2_ref[...]) + be2_ref[...])
    cm = _silu(_dot(m, wc1_ref[...]) + bc1_ref[...])
    s = _dot(cm, wc2_ref[...])
    lane = lax.broadcasted_iota(jnp.int32, (1, 16), 1)
    one3 = jnp.where(lane == 3, 1.0, 0.0).astype(jnp.float32)
    tr = cdiff * s + one3
    o_ref[...] = jnp.concatenate([m, tr], axis=1)


def _edge(A, B, EA, p):
    grid = (E_PAD // BLK_E,)
    wrad = p['We1'][2 * HID].reshape(1, HID)
    wea = p['We1'][2 * HID + 1:]
    return pl.pallas_call(
        _edge_body,
        grid=grid,
        in_specs=[
            pl.BlockSpec((BLK_E, GW), lambda i: (i, 0)),
            pl.BlockSpec((BLK_E, GW), lambda i: (i, 0)),
            pl.BlockSpec((BLK_E, EDGE_DIM), lambda i: (i, 0)),
            pl.BlockSpec((1, HID), lambda i: (0, 0)),
            pl.BlockSpec((EDGE_DIM, HID), lambda i: (0, 0)),
            pl.BlockSpec((1, HID), lambda i: (0, 0)),
            pl.BlockSpec((HID, HID), lambda i: (0, 0)),
            pl.BlockSpec((1, HID), lambda i: (0, 0)),
            pl.BlockSpec((HID, HID), lambda i: (0, 0)),
            pl.BlockSpec((1, HID), lambda i: (0, 0)),
            pl.BlockSpec((HID, 1), lambda i: (0, 0)),
        ],
        out_specs=pl.BlockSpec((BLK_E, TW), lambda i: (i, 0)),
        out_shape=jax.ShapeDtypeStruct((E_PAD, TW), jnp.float32),
    )(A, B, EA, wrad, wea, p['be1'].reshape(1, HID), p['We2'],
      p['be2'].reshape(1, HID), p['Wc1'], p['bc1'].reshape(1, HID), p['Wc2'])


def _node_body(h_ref, c_ref, agg_ref, wa_ref, wb_ref, bn1_ref, wn2_ref,
               bn2_ref, ho_ref, co_ref):
    h = h_ref[...]
    agg = agg_ref[...]
    am = agg[:, :HID]
    at = agg[:, HID:TW]
    cnt = at[:, 3:4]
    recip = 1.0 / jnp.maximum(cnt, 1.0)
    lane = lax.broadcasted_iota(jnp.int32, (1, 16), 1)
    mask3 = jnp.where(lane < 3, 1.0, 0.0).astype(jnp.float32)
    co_ref[...] = c_ref[...] + at * recip * mask3
    t = _silu(_dot(h, wa_ref[...]) + _dot(am, wb_ref[...]) + bn1_ref[...])
    ho_ref[...] = h + _dot(t, wn2_ref[...]) + bn2_ref[...]


def _node(h, coordp, AGG, p):
    grid = (N_PAD // BLK_N,)
    return pl.pallas_call(
        _node_body,
        grid=grid,
        in_specs=[
            pl.BlockSpec((BLK_N, HID), lambda i: (i, 0)),
            pl.BlockSpec((BLK_N, 16), lambda i: (i, 0)),
            pl.BlockSpec((BLK_N, TW), lambda i: (i, 0)),
            pl.BlockSpec((HID, HID), lambda i: (0, 0)),
            pl.BlockSpec((HID, HID), lambda i: (0, 0)),
            pl.BlockSpec((1, HID), lambda i: (0, 0)),
            pl.BlockSpec((HID, HID), lambda i: (0, 0)),
            pl.BlockSpec((1, HID), lambda i: (0, 0)),
        ],
        out_specs=[
            pl.BlockSpec((BLK_N, HID), lambda i: (i, 0)),
            pl.BlockSpec((BLK_N, 16), lambda i: (i, 0)),
        ],
        out_shape=[
            jax.ShapeDtypeStruct((N_PAD, HID), jnp.float32),
            jax.ShapeDtypeStruct((N_PAD, 16), jnp.float32),
        ],
    )(h, coordp, AGG, p['Wn1'][:HID], p['Wn1'][HID:],
      p['bn1'].reshape(1, HID), p['Wn2'], p['bn2'].reshape(1, HID))


def _post_body(h_ref, w_ref, b_ref, o_ref):
    o_ref[...] = _dot(h_ref[...], w_ref[...]) + b_ref[...]


def _post(h, w, b):
    grid = (N_PAD // BLK_N,)
    return pl.pallas_call(
        _post_body,
        grid=grid,
        in_specs=[
            pl.BlockSpec((BLK_N, HID), lambda i: (i, 0)),
            pl.BlockSpec((HID, HID), lambda i: (0, 0)),
            pl.BlockSpec((1, HID), lambda i: (0, 0)),
        ],
        out_specs=pl.BlockSpec((BLK_N, HID), lambda i: (i, 0)),
        out_shape=jax.ShapeDtypeStruct((N_PAD, HID), jnp.float32),
    )(h, w, b.reshape(1, HID))


def _gat_prep_body(x_ref, wl_ref, bl_ref, br_ref, att_ref, xl_ref, e_ref,
                   mx_ref):
    i = pl.program_id(0)
    xl = _dot(x_ref[...], wl_ref[...]) + bl_ref[...]
    xl_ref[...] = xl
    e = _dot(jax.nn.leaky_relu(xl + br_ref[...], 0.2), att_ref[...])
    e_ref[...] = e

    @pl.when(i == 0)
    def _init():
        mx_ref[...] = jnp.full_like(mx_ref[...], -jnp.inf)

    mx_ref[...] = jnp.maximum(mx_ref[...], jnp.max(e))


def _gat_prep(x, gp, blk):
    m = x.shape[0]
    grid = (m // blk,)
    return pl.pallas_call(
        _gat_prep_body,
        grid=grid,
        in_specs=[
            pl.BlockSpec((blk, HID), lambda i: (i, 0)),
            pl.BlockSpec((HID, HID), lambda i: (0, 0)),
            pl.BlockSpec((1, HID), lambda i: (0, 0)),
            pl.BlockSpec((1, HID), lambda i: (0, 0)),
            pl.BlockSpec((HID, 1), lambda i: (0, 0)),
        ],
        out_specs=[
            pl.BlockSpec((blk, HID), lambda i: (i, 0)),
            pl.BlockSpec((blk, 1), lambda i: (i, 0)),
            pl.BlockSpec((1, 1), lambda i: (0, 0)),
        ],
        out_shape=[
            jax.ShapeDtypeStruct((m, HID), jnp.float32),
            jax.ShapeDtypeStruct((m, 1), jnp.float32),
            jax.ShapeDtypeStruct((1, 1), jnp.float32),
        ],
    )(x, gp['Wl'], gp['bl'].reshape(1, HID), gp['br'].reshape(1, HID),
      gp['att'].reshape(HID, 1))


def _gat_w_body(xl_ref, e_ref, mx_ref, o_ref):
    w = jnp.exp(e_ref[...] - mx_ref[...])
    lane = lax.broadcasted_iota(jnp.int32, (1, 16), 1)
    one0 = jnp.where(lane == 0, 1.0, 0.0).astype(jnp.float32)
    o_ref[...] = jnp.concatenate([w * xl_ref[...], w * one0], axis=1)


def _gat_w(xl, e, mx, blk):
    m = xl.shape[0]
    grid = (m // blk,)
    return pl.pallas_call(
        _gat_w_body,
        grid=grid,
        in_specs=[
            pl.BlockSpec((blk, HID), lambda i: (i, 0)),
            pl.BlockSpec((blk, 1), lambda i: (i, 0)),
            pl.BlockSpec((1, 1), lambda i: (0, 0)),
        ],
        out_specs=pl.BlockSpec((blk, TW), lambda i: (i, 0)),
        out_shape=jax.ShapeDtypeStruct((m, TW), jnp.float32),
    )(xl, e, mx)


def _gat_out_body(agg_ref, mx_ref, c0_ref, bl_ref, bias_ref, o_ref):
    wself = jnp.exp(c0_ref[...] - mx_ref[...])
    numer = agg_ref[:, :HID] + wself * bl_ref[...]
    denom = agg_ref[:, HID:HID + 1] + wself + 1e-16
    o_ref[...] = numer / denom + bias_ref[...]


def _gat_out(agg, mx, c0, gp, blk):
    c = agg.shape[0]
    grid = (c // blk,)
    return pl.pallas_call(
        _gat_out_body,
        grid=grid,
        in_specs=[
            pl.BlockSpec((blk, TW), lambda i: (i, 0)),
            pl.BlockSpec((1, 1), lambda i: (0, 0)),
            pl.BlockSpec((1, 1), lambda i: (0, 0)),
            pl.BlockSpec((1, HID), lambda i: (0, 0)),
            pl.BlockSpec((1, HID), lambda i: (0, 0)),
        ],
        out_specs=pl.BlockSpec((blk, HID), lambda i: (i, 0)),
        out_shape=jax.ShapeDtypeStruct((c, HID), jnp.float32),
    )(agg, mx, c0, gp['bl'].reshape(1, HID), gp['bias'].reshape(1, HID))


_MESH = plsc.VectorSubcoreMesh(core_axis_name="c", subcore_axis_name="s")
NC = 2
NS = 16


def _sc_gather(table, idx2d, width):
    blks = idx2d.shape[0]
    per_w = blks // (NC * NS)
    outer = per_w // 8

    def body(table_h, idx_h, out_h, idx_v, rows_v, sem):
        c = lax.axis_index("c")
        s = lax.axis_index("s")
        wid = s * NC + c
        base = wid * per_w

        def it(g, carry):
            b0 = base + g * 8
            pltpu.sync_copy(idx_h.at[pl.ds(b0, 8)], idx_v)
            for half in range(2):
                cps = [
                    pltpu.async_copy(table_h.at[idx_v.at[half * 4 + j]],
                                     rows_v.at[pl.ds(j * 128, 128)], sem)
                    for j in range(4)
                ]
                for cp in cps:
                    cp.wait()
                pltpu.sync_copy(
                    rows_v, out_h.at[pl.ds((b0 + half * 4) * 128, 512)])
            return carry

        lax.fori_loop(0, outer, it, 0)

    f = pl.kernel(
        body,
        out_type=jax.ShapeDtypeStruct((blks * 128, width), jnp.float32),
        mesh=_MESH,
        scratch_types=[
            pltpu.VMEM((8, 128), jnp.int32),
            pltpu.VMEM((512, width), jnp.float32),
            pltpu.SemaphoreType.DMA,
        ],
    )
    return f(table, idx2d)


def _sc_scatter_add(S3, idx2d, zeros, acc_rows, dump_chunk):
    blks = idx2d.shape[0]
    per_tile = blks // NS
    K = 4
    outer = per_tile // K
    rows_pt = acc_rows // NS
    n_dump = rows_pt // dump_chunk

    def body(S_h, idx_h, z_h, out_h, idx_v, vals_v, dump_v, acc_sh):
        c = lax.axis_index("c")
        s = lax.axis_index("s")
        pltpu.sync_copy(z_h.at[pl.ds(s * rows_pt, rows_pt)],
                        acc_sh.at[pl.ds(s * rows_pt, rows_pt)])
        plsc.subcore_barrier()
        base = s * per_tile

        def it(g, carry):
            b0 = base + g * K
            pltpu.sync_copy(idx_h.at[pl.ds(b0, K)], idx_v)
            pltpu.sync_copy(S_h.at[c, pl.ds(b0 * 128, K * 128)], vals_v)
            for j in range(K):
                pltpu.sync_copy(vals_v.at[pl.ds(j * 128, 128)],
                                acc_sh.at[idx_v.at[j]], add=True)
            return carry

        lax.fori_loop(0, outer, it, 0)
        plsc.subcore_barrier()

        def dmp(d, carry):
            r0 = s * rows_pt + d * dump_chunk
            pltpu.sync_copy(acc_sh.at[pl.ds(r0, dump_chunk)], dump_v)
            pltpu.sync_copy(dump_v, out_h.at[c, pl.ds(r0, dump_chunk)])
            return carry

        lax.fori_loop(0, n_dump, dmp, 0)

    f = pl.kernel(
        body,
        out_type=jax.ShapeDtypeStruct((2, acc_rows, 40), jnp.float32),
        mesh=_MESH,
        scratch_types=[
            pltpu.VMEM((K, 128), jnp.int32),
            pltpu.VMEM((K * 128, 40), jnp.float32),
            pltpu.VMEM((dump_chunk, 40), jnp.float32),
            pltpu.VMEM_SHARED((acc_rows, 40), jnp.float32),
        ],
    )
    return f(S3, idx2d, zeros)


def _gather_rows(table, idx):
    return jnp.take(table, idx, axis=0)


def _scatter_add(values, ids, num_segments):
    return jax.ops.segment_sum(values, ids, num_segments=num_segments)


@jax.jit
def _run(x, pos, edge_index, edge_attr, batch_lig, batch_pro,
         lin_W, lin_b, W_in, b_in, layers, W_out, b_out, gat_lig, gat_pro):
    f32 = jnp.float32
    xp = jnp.zeros((N_PAD, NODE_DIM), f32).at[:N].set(x)
    coordp = jnp.zeros((N_PAD, 16), f32).at[:N, :3].set(pos)
    row = edge_index[0]
    col = edge_index[1]
    rowp = jnp.full((E_PAD,), N, jnp.int32).at[:E].set(row)
    colp = jnp.full((E_PAD,), N, jnp.int32).at[:E].set(col)
    row2d = rowp.reshape(E_PAD // 128, 128)
    col2d = colp.reshape(E_PAD // 128, 128)
    eap = jnp.zeros((E_PAD, EDGE_DIM), f32).at[:E].set(edge_attr)

    h = _mlp_in(xp, lin_W, lin_b, W_in, b_in)
    for p in layers:
        T1, T2 = _prep(h, coordp, p['We1'][:HID], p['We1'][HID:2 * HID])
        A = _sc_gather(T1, row2d, GW)
        B = _sc_gather(T2, col2d, GW)
        S = _edge(A, B, eap, p)
        AGG = _scatter_add(S, rowp, N_PAD)
        h, coordp = _node(h, coordp, AGG, p)

    hout = _post(h, W_out, b_out)

    def gat(xa, batch, c, blk_a, blk_c, gp):
        xl, e, mx = _gat_prep(xa, gp, blk_a)
        c0 = jnp.dot(jax.nn.leaky_relu(gp['bl'] + gp['br'], 0.2),
                     gp['att']).reshape(1, 1)
        mx = jnp.maximum(mx, c0)
        S = _gat_w(xl, e, mx, blk_a)
        agg = _scatter_add(S, batch, c)
        return _gat_out(agg, mx, c0, gp, blk_c)

    x_lig_cg = gat(hout[:L], batch_lig, C_LIG, 1000, 1000, gat_lig)
    x_pro_cg = gat(hout[L:L + P], batch_pro, C_PRO, 1600, 1000, gat_pro)
    pos_lig = coordp[:L, :3]
    pos_pro = coordp[L:L + P, :3]
    return x_lig_cg, x_pro_cg, pos_lig, pos_pro


def kernel(x, pos, edge_index, edge_attr, batch_lig, batch_pro,
           lin_W, lin_b, W_in, b_in, layers, W_out, b_out, gat_lig, gat_pro):
    return _run(x, pos, edge_index, edge_attr, batch_lig, batch_pro,
                lin_W, lin_b, W_in, b_in, layers, W_out, b_out,
                gat_lig, gat_pro)


# speedup vs baseline: 2.0441x; 1.0437x over previous
import functools
import jax
import jax.numpy as jnp
from jax import lax
from jax.experimental import pallas as pl
from jax.experimental.pallas import tpu as pltpu
from jax.experimental.pallas import tpu_sc as plsc

N = 50000
E = 800000
L = 10000
P = 40000
NODE_DIM = 128
HID = 64
EDGE_DIM = 4
C_LIG = 2000
C_PRO = 8000

N_PAD = 50176
BLK_N = 1568
E_PAD = 819200
BLK_E = 4096
TW = 80
GW = 128


def _silu(v):
    return v * jax.nn.sigmoid(v)


def _dot(a, b):
    return jnp.dot(a, b, preferred_element_type=jnp.float32)


def _mlp_in_body(x_ref, w1_ref, b1_ref, w2_ref, b2_ref, o_ref):
    h = _silu(_dot(x_ref[...], w1_ref[...]) + b1_ref[...])
    o_ref[...] = _dot(h, w2_ref[...]) + b2_ref[...]


def _mlp_in(x, w1, b1, w2, b2):
    grid = (N_PAD // BLK_N,)
    return pl.pallas_call(
        _mlp_in_body,
        grid=grid,
        in_specs=[
            pl.BlockSpec((BLK_N, NODE_DIM), lambda i: (i, 0)),
            pl.BlockSpec((NODE_DIM, HID), lambda i: (0, 0)),
            pl.BlockSpec((1, HID), lambda i: (0, 0)),
            pl.BlockSpec((HID, HID), lambda i: (0, 0)),
            pl.BlockSpec((1, HID), lambda i: (0, 0)),
        ],
        out_specs=pl.BlockSpec((BLK_N, HID), lambda i: (i, 0)),
        out_shape=jax.ShapeDtypeStruct((N_PAD, HID), jnp.float32),
    )(x, w1, b1.reshape(1, HID), w2, b2.reshape(1, HID))


def _prep_body(h_ref, c_ref, wr_ref, wc_ref, t1_ref, t2_ref):
    h = h_ref[...]
    c = c_ref[...]
    z = jnp.zeros((h.shape[0], GW - TW), jnp.float32)
    t1_ref[...] = jnp.concatenate([_dot(h, wr_ref[...]), c, z], axis=1)
    t2_ref[...] = jnp.concatenate([_dot(h, wc_ref[...]), c, z], axis=1)


def _prep(h, coordp, wr, wc):
    grid = (N_PAD // BLK_N,)
    return pl.pallas_call(
        _prep_body,
        grid=grid,
        in_specs=[
            pl.BlockSpec((BLK_N, HID), lambda i: (i, 0)),
            pl.BlockSpec((BLK_N, 16), lambda i: (i, 0)),
            pl.BlockSpec((HID, HID), lambda i: (0, 0)),
            pl.BlockSpec((HID, HID), lambda i: (0, 0)),
        ],
        out_specs=[
            pl.BlockSpec((BLK_N, GW), lambda i: (i, 0)),
            pl.BlockSpec((BLK_N, GW), lambda i: (i, 0)),
        ],
        out_shape=[
            jax.ShapeDtypeStruct((N_PAD, GW), jnp.float32),
            jax.ShapeDtypeStruct((N_PAD, GW), jnp.float32),
        ],
    )(h, coordp, wr, wc)


def _edge_body(a_ref, b_ref, ea_ref, wrad_ref, wea_ref, be1_ref, we2_ref,
               be2_ref, wc1_ref, bc1_ref, wc2_ref, o_ref):
    a = a_ref[...]
    b = b_ref[...]
    af = a[:, :HID]
    bf = b[:, :HID]
    ac = a[:, HID:TW]
    bc = b[:, HID:TW]
    cdiff = ac - bc
    radial = jnp.sum(cdiff * cdiff, axis=1, keepdims=True)
    z = af + bf + radial * wrad_ref[...] + _dot(ea_ref[...], wea_ref[...]) \
        + be1_ref[...]
    m1 = _silu(z)
    m = _silu(_dot(m1, we2_ref[...]) + be2_ref[...])
    cm = _silu(_dot(m, wc1_ref[...]) + bc1_ref[...])
    s = _dot(cm, wc2_ref[...])
    lane = lax.broadcasted_iota(jnp.int32, (1, 16), 1)
    one3 = jnp.where(lane == 3, 1.0, 0.0).astype(jnp.float32)
    tr = cdiff * s + one3
    o_ref[...] = jnp.concatenate([m, tr], axis=1)


def _edge(A, B, EA, p):
    grid = (E_PAD // BLK_E,)
    wrad = p['We1'][2 * HID].reshape(1, HID)
    wea = p['We1'][2 * HID + 1:]
    return pl.pallas_call(
        _edge_body,
        grid=grid,
        in_specs=[
            pl.BlockSpec((BLK_E, GW), lambda i: (i, 0)),
            pl.BlockSpec((BLK_E, GW), lambda i: (i, 0)),
            pl.BlockSpec((BLK_E, EDGE_DIM), lambda i: (i, 0)),
            pl.BlockSpec((1, HID), lambda i: (0, 0)),
            pl.BlockSpec((EDGE_DIM, HID), lambda i: (0, 0)),
            pl.BlockSpec((1, HID), lambda i: (0, 0)),
            pl.BlockSpec((HID, HID), lambda i: (0, 0)),
            pl.BlockSpec((1, HID), lambda i: (0, 0)),
            pl.BlockSpec((HID, HID), lambda i: (0, 0)),
            pl.BlockSpec((1, HID), lambda i: (0, 0)),
            pl.BlockSpec((HID, 1), lambda i: (0, 0)),
        ],
        out_specs=pl.BlockSpec((BLK_E, TW), lambda i: (i, 0)),
        out_shape=jax.ShapeDtypeStruct((E_PAD, TW), jnp.float32),
    )(A, B, EA, wrad, wea, p['be1'].reshape(1, HID), p['We2'],
      p['be2'].reshape(1, HID), p['Wc1'], p['bc1'].reshape(1, HID), p['Wc2'])


def _node_body(h_ref, c_ref, agg_ref, wa_ref, wb_ref, bn1_ref, wn2_ref,
               bn2_ref, ho_ref, co_ref):
    h = h_ref[...]
    agg = agg_ref[...]
    am = agg[:, :HID]
    at = agg[:, HID:TW]
    cnt = at[:, 3:4]
    recip = 1.0 / jnp.maximum(cnt, 1.0)
    lane = lax.broadcasted_iota(jnp.int32, (1, 16), 1)
    mask3 = jnp.where(lane < 3, 1.0, 0.0).astype(jnp.float32)
    co_ref[...] = c_ref[...] + at * recip * mask3
    t = _silu(_dot(h, wa_ref[...]) + _dot(am, wb_ref[...]) + bn1_ref[...])
    ho_ref[...] = h + _dot(t, wn2_ref[...]) + bn2_ref[...]


def _node(h, coordp, AGG, p):
    grid = (N_PAD // BLK_N,)
    return pl.pallas_call(
        _node_body,
        grid=grid,
        in_specs=[
            pl.BlockSpec((BLK_N, HID), lambda i: (i, 0)),
            pl.BlockSpec((BLK_N, 16), lambda i: (i, 0)),
            pl.BlockSpec((BLK_N, TW), lambda i: (i, 0)),
            pl.BlockSpec((HID, HID), lambda i: (0, 0)),
            pl.BlockSpec((HID, HID), lambda i: (0, 0)),
            pl.BlockSpec((1, HID), lambda i: (0, 0)),
            pl.BlockSpec((HID, HID), lambda i: (0, 0)),
            pl.BlockSpec((1, HID), lambda i: (0, 0)),
        ],
        out_specs=[
            pl.BlockSpec((BLK_N, HID), lambda i: (i, 0)),
            pl.BlockSpec((BLK_N, 16), lambda i: (i, 0)),
        ],
        out_shape=[
            jax.ShapeDtypeStruct((N_PAD, HID), jnp.float32),
            jax.ShapeDtypeStruct((N_PAD, 16), jnp.float32),
        ],
    )(h, coordp, AGG, p['Wn1'][:HID], p['Wn1'][HID:],
      p['bn1'].reshape(1, HID), p['Wn2'], p['bn2'].reshape(1, HID))


def _post_body(h_ref, w_ref, b_ref, o_ref):
    o_ref[...] = _dot(h_ref[...], w_ref[...]) + b_ref[...]


def _post(h, w, b):
    grid = (N_PAD // BLK_N,)
    return pl.pallas_call(
        _post_body,
        grid=grid,
        in_specs=[
            pl.BlockSpec((BLK_N, HID), lambda i: (i, 0)),
            pl.BlockSpec((HID, HID), lambda i: (0, 0)),
            pl.BlockSpec((1, HID), lambda i: (0, 0)),
        ],
        out_specs=pl.BlockSpec((BLK_N, HID), lambda i: (i, 0)),
        out_shape=jax.ShapeDtypeStruct((N_PAD, HID), jnp.float32),
    )(h, w, b.reshape(1, HID))


def _gat_prep_body(x_ref, wl_ref, bl_ref, br_ref, att_ref, xl_ref, e_ref,
                   mx_ref):
    i = pl.program_id(0)
    xl = _dot(x_ref[...], wl_ref[...]) + bl_ref[...]
    xl_ref[...] = xl
    e = _dot(jax.nn.leaky_relu(xl + br_ref[...], 0.2), att_ref[...])
    e_ref[...] = e

    @pl.when(i == 0)
    def _init():
        mx_ref[...] = jnp.full_like(mx_ref[...], -jnp.inf)

    mx_ref[...] = jnp.maximum(mx_ref[...], jnp.max(e))


def _gat_prep(x, gp, blk):
    m = x.shape[0]
    grid = (m // blk,)
    return pl.pallas_call(
        _gat_prep_body,
        grid=grid,
        in_specs=[
            pl.BlockSpec((blk, HID), lambda i: (i, 0)),
            pl.BlockSpec((HID, HID), lambda i: (0, 0)),
            pl.BlockSpec((1, HID), lambda i: (0, 0)),
            pl.BlockSpec((1, HID), lambda i: (0, 0)),
            pl.BlockSpec((HID, 1), lambda i: (0, 0)),
        ],
        out_specs=[
            pl.BlockSpec((blk, HID), lambda i: (i, 0)),
            pl.BlockSpec((blk, 1), lambda i: (i, 0)),
            pl.BlockSpec((1, 1), lambda i: (0, 0)),
        ],
        out_shape=[
            jax.ShapeDtypeStruct((m, HID), jnp.float32),
            jax.ShapeDtypeStruct((m, 1), jnp.float32),
            jax.ShapeDtypeStruct((1, 1), jnp.float32),
        ],
    )(x, gp['Wl'], gp['bl'].reshape(1, HID), gp['br'].reshape(1, HID),
      gp['att'].reshape(HID, 1))


def _gat_w_body(xl_ref, e_ref, mx_ref, o_ref):
    w = jnp.exp(e_ref[...] - mx_ref[...])
    lane = lax.broadcasted_iota(jnp.int32, (1, 16), 1)
    one0 = jnp.where(lane == 0, 1.0, 0.0).astype(jnp.float32)
    o_ref[...] = jnp.concatenate([w * xl_ref[...], w * one0], axis=1)


def _gat_w(xl, e, mx, blk):
    m = xl.shape[0]
    grid = (m // blk,)
    return pl.pallas_call(
        _gat_w_body,
        grid=grid,
        in_specs=[
            pl.BlockSpec((blk, HID), lambda i: (i, 0)),
            pl.BlockSpec((blk, 1), lambda i: (i, 0)),
            pl.BlockSpec((1, 1), lambda i: (0, 0)),
        ],
        out_specs=pl.BlockSpec((blk, TW), lambda i: (i, 0)),
        out_shape=jax.ShapeDtypeStruct((m, TW), jnp.float32),
    )(xl, e, mx)


def _gat_out_body(agg_ref, mx_ref, c0_ref, bl_ref, bias_ref, o_ref):
    wself = jnp.exp(c0_ref[...] - mx_ref[...])
    numer = agg_ref[:, :HID] + wself * bl_ref[...]
    denom = agg_ref[:, HID:HID + 1] + wself + 1e-16
    o_ref[...] = numer / denom + bias_ref[...]


def _gat_out(agg, mx, c0, gp, blk):
    c = agg.shape[0]
    grid = (c // blk,)
    return pl.pallas_call(
        _gat_out_body,
        grid=grid,
        in_specs=[
            pl.BlockSpec((blk, TW), lambda i: (i, 0)),
            pl.BlockSpec((1, 1), lambda i: (0, 0)),
            pl.BlockSpec((1, 1), lambda i: (0, 0)),
            pl.BlockSpec((1, HID), lambda i: (0, 0)),
            pl.BlockSpec((1, HID), lambda i: (0, 0)),
        ],
        out_specs=pl.BlockSpec((blk, HID), lambda i: (i, 0)),
        out_shape=jax.ShapeDtypeStruct((c, HID), jnp.float32),
    )(agg, mx, c0, gp['bl'].reshape(1, HID), gp['bias'].reshape(1, HID))


_MESH = plsc.VectorSubcoreMesh(core_axis_name="c", subcore_axis_name="s")
NC = 2
NS = 16


def _sc_gather(table, idx2d, width):
    blks = idx2d.shape[0]
    per_w = blks // (NC * NS)

    def body(table_h, idx_h, out_h, idx_all, rows_v, g0, g1, s0, s1):
        c = lax.axis_index("c")
        s = lax.axis_index("s")
        wid = s * NC + c
        base = wid * per_w
        gsem = (g0, g1)
        ssem = (s0, s1)
        pltpu.sync_copy(idx_h.at[pl.ds(base, per_w)], idx_all)

        def gath(j, b):
            return pltpu.async_copy(table_h.at[idx_all.at[j]],
                                    rows_v.at[b], gsem[b])

        def stor(j, b):
            return pltpu.async_copy(rows_v.at[b],
                                    out_h.at[pl.ds((base + j) * 128, 128)],
                                    ssem[b])

        gath(0, 0)
        gath(1, 1)

        def group(gg, carry):
            for b in range(2):
                j = gg * 2 + b
                pltpu.make_async_copy(table_h.at[idx_all.at[j]],
                                      rows_v.at[b], gsem[b]).wait()
                stor(j, b)
                pltpu.make_async_copy(rows_v.at[b],
                                      out_h.at[pl.ds((base + j) * 128, 128)],
                                      ssem[b]).wait()

                @pl.when(j + 2 < per_w)
                def _next():
                    gath(j + 2, b)
            return carry

        lax.fori_loop(0, per_w // 2, group, 0)

    f = pl.kernel(
        body,
        out_type=jax.ShapeDtypeStruct((blks * 128, width), jnp.float32),
        mesh=_MESH,
        scratch_types=[
            pltpu.VMEM((per_w, 128), jnp.int32),
            pltpu.VMEM((2, 128, width), jnp.float32),
            pltpu.SemaphoreType.DMA,
            pltpu.SemaphoreType.DMA,
            pltpu.SemaphoreType.DMA,
            pltpu.SemaphoreType.DMA,
        ],
    )
    return f(table, idx2d)


def _sc_scatter_add(S3, idx2d, zeros, acc_rows, dump_chunk):
    blks = idx2d.shape[0]
    per_tile = blks // NS
    K = 4
    outer = per_tile // K
    rows_pt = acc_rows // NS
    n_dump = rows_pt // dump_chunk

    def body(S_h, idx_h, z_h, out_h, idx_v, vals_v, dump_v, acc_sh):
        c = lax.axis_index("c")
        s = lax.axis_index("s")
        pltpu.sync_copy(z_h.at[pl.ds(s * rows_pt, rows_pt)],
                        acc_sh.at[pl.ds(s * rows_pt, rows_pt)])
        plsc.subcore_barrier()
        base = s * per_tile

        def it(g, carry):
            b0 = base + g * K
            pltpu.sync_copy(idx_h.at[pl.ds(b0, K)], idx_v)
            pltpu.sync_copy(S_h.at[c, pl.ds(b0 * 128, K * 128)], vals_v)
            for j in range(K):
                pltpu.sync_copy(vals_v.at[pl.ds(j * 128, 128)],
                                acc_sh.at[idx_v.at[j]], add=True)
            return carry

        lax.fori_loop(0, outer, it, 0)
        plsc.subcore_barrier()

        def dmp(d, carry):
            r0 = s * rows_pt + d * dump_chunk
            pltpu.sync_copy(acc_sh.at[pl.ds(r0, dump_chunk)], dump_v)
            pltpu.sync_copy(dump_v, out_h.at[c, pl.ds(r0, dump_chunk)])
            return carry

        lax.fori_loop(0, n_dump, dmp, 0)

    f = pl.kernel(
        body,
        out_type=jax.ShapeDtypeStruct((2, acc_rows, 40), jnp.float32),
        mesh=_MESH,
        scratch_types=[
            pltpu.VMEM((K, 128), jnp.int32),
            pltpu.VMEM((K * 128, 40), jnp.float32),
            pltpu.VMEM((dump_chunk, 40), jnp.float32),
            pltpu.VMEM_SHARED((acc_rows, 40), jnp.float32),
        ],
    )
    return f(S3, idx2d, zeros)


def _gather_rows(table, idx):
    return jnp.take(table, idx, axis=0)


def _scatter_add(values, ids, num_segments):
    return jax.ops.segment_sum(values, ids, num_segments=num_segments)


@jax.jit
def _run(x, pos, edge_index, edge_attr, batch_lig, batch_pro,
         lin_W, lin_b, W_in, b_in, layers, W_out, b_out, gat_lig, gat_pro):
    f32 = jnp.float32
    xp = jnp.zeros((N_PAD, NODE_DIM), f32).at[:N].set(x)
    coordp = jnp.zeros((N_PAD, 16), f32).at[:N, :3].set(pos)
    row = edge_index[0]
    col = edge_index[1]
    rowp = jnp.full((E_PAD,), N, jnp.int32).at[:E].set(row)
    colp = jnp.full((E_PAD,), N, jnp.int32).at[:E].set(col)
    row2d = rowp.reshape(E_PAD // 128, 128)
    col2d = colp.reshape(E_PAD // 128, 128)
    eap = jnp.zeros((E_PAD, EDGE_DIM), f32).at[:E].set(edge_attr)

    h = _mlp_in(xp, lin_W, lin_b, W_in, b_in)
    for p in layers:
        T1, T2 = _prep(h, coordp, p['We1'][:HID], p['We1'][HID:2 * HID])
        A = _sc_gather(T1, row2d, GW)
        B = _sc_gather(T2, col2d, GW)
        S = _edge(A, B, eap, p)
        AGG = _scatter_add(S, rowp, N_PAD)
        h, coordp = _node(h, coordp, AGG, p)

    hout = _post(h, W_out, b_out)

    def gat(xa, batch, c, blk_a, blk_c, gp):
        xl, e, mx = _gat_prep(xa, gp, blk_a)
        c0 = jnp.dot(jax.nn.leaky_relu(gp['bl'] + gp['br'], 0.2),
                     gp['att']).reshape(1, 1)
        mx = jnp.maximum(mx, c0)
        S = _gat_w(xl, e, mx, blk_a)
        agg = _scatter_add(S, batch, c)
        return _gat_out(agg, mx, c0, gp, blk_c)

    x_lig_cg = gat(hout[:L], batch_lig, C_LIG, 1000, 1000, gat_lig)
    x_pro_cg = gat(hout[L:L + P], batch_pro, C_PRO, 1600, 1000, gat_pro)
    pos_lig = coordp[:L, :3]
    pos_pro = coordp[L:L + P, :3]
    return x_lig_cg, x_pro_cg, pos_lig, pos_pro


def kernel(x, pos, edge_index, edge_attr, batch_lig, batch_pro,
           lin_W, lin_b, W_in, b_in, layers, W_out, b_out, gat_lig, gat_pro):
    return _run(x, pos, edge_index, edge_attr, batch_lig, batch_pro,
                lin_W, lin_b, W_in, b_in, layers, W_out, b_out,
                gat_lig, gat_pro)


# speedup vs baseline: 2.4294x; 1.1885x over previous
import functools
import jax
import jax.numpy as jnp
from jax import lax
from jax.experimental import pallas as pl
from jax.experimental.pallas import tpu as pltpu
from jax.experimental.pallas import tpu_sc as plsc

N = 50000
E = 800000
L = 10000
P = 40000
NODE_DIM = 128
HID = 64
EDGE_DIM = 4
C_LIG = 2000
C_PRO = 8000

N_PAD = 50176
BLK_N = 1568
E_PAD = 819200
BLK_E = 4096
TW = 80
GW = 128


def _silu(v):
    return v * jax.nn.sigmoid(v)


def _dot(a, b):
    return jnp.dot(a, b, preferred_element_type=jnp.float32)


def _mlp_in_body(x_ref, w1_ref, b1_ref, w2_ref, b2_ref, o_ref):
    h = _silu(_dot(x_ref[...], w1_ref[...]) + b1_ref[...])
    o_ref[...] = _dot(h, w2_ref[...]) + b2_ref[...]


def _mlp_in(x, w1, b1, w2, b2):
    grid = (N_PAD // BLK_N,)
    return pl.pallas_call(
        _mlp_in_body,
        grid=grid,
        in_specs=[
            pl.BlockSpec((BLK_N, NODE_DIM), lambda i: (i, 0)),
            pl.BlockSpec((NODE_DIM, HID), lambda i: (0, 0)),
            pl.BlockSpec((1, HID), lambda i: (0, 0)),
            pl.BlockSpec((HID, HID), lambda i: (0, 0)),
            pl.BlockSpec((1, HID), lambda i: (0, 0)),
        ],
        out_specs=pl.BlockSpec((BLK_N, HID), lambda i: (i, 0)),
        out_shape=jax.ShapeDtypeStruct((N_PAD, HID), jnp.float32),
    )(x, w1, b1.reshape(1, HID), w2, b2.reshape(1, HID))


def _prep_body(h_ref, c_ref, wr_ref, wc_ref, t1_ref, t2_ref):
    h = h_ref[...]
    c = c_ref[...]
    z = jnp.zeros((h.shape[0], GW - TW), jnp.float32)
    t1_ref[...] = jnp.concatenate([_dot(h, wr_ref[...]), c, z], axis=1)
    t2_ref[...] = jnp.concatenate([_dot(h, wc_ref[...]), -c, z], axis=1)


def _prep(h, coordp, wr, wc):
    grid = (N_PAD // BLK_N,)
    return pl.pallas_call(
        _prep_body,
        grid=grid,
        in_specs=[
            pl.BlockSpec((BLK_N, HID), lambda i: (i, 0)),
            pl.BlockSpec((BLK_N, 16), lambda i: (i, 0)),
            pl.BlockSpec((HID, HID), lambda i: (0, 0)),
            pl.BlockSpec((HID, HID), lambda i: (0, 0)),
        ],
        out_specs=[
            pl.BlockSpec((BLK_N, GW), lambda i: (i, 0)),
            pl.BlockSpec((BLK_N, GW), lambda i: (i, 0)),
        ],
        out_shape=[
            jax.ShapeDtypeStruct((N_PAD, GW), jnp.float32),
            jax.ShapeDtypeStruct((N_PAD, GW), jnp.float32),
        ],
    )(h, coordp, wr, wc)


def _edge_body(a_ref, ea_ref, wrad_ref, wea_ref, be1_ref, we2_ref,
               be2_ref, wc1_ref, bc1_ref, wc2_ref, o_ref):
    a = a_ref[...]
    af = a[:, :HID]
    cdiff = a[:, HID:TW]
    radial = jnp.sum(cdiff * cdiff, axis=1, keepdims=True)
    z = af + radial * wrad_ref[...] + _dot(ea_ref[...], wea_ref[...]) \
        + be1_ref[...]
    m1 = _silu(z)
    m = _silu(_dot(m1, we2_ref[...]) + be2_ref[...])
    cm = _silu(_dot(m, wc1_ref[...]) + bc1_ref[...])
    s = _dot(cm, wc2_ref[...])
    lane = lax.broadcasted_iota(jnp.int32, (1, 16), 1)
    one3 = jnp.where(lane == 3, 1.0, 0.0).astype(jnp.float32)
    tr = cdiff * s + one3
    o_ref[...] = jnp.concatenate([m, tr], axis=1)


def _edge(A, EA, p):
    grid = (E_PAD // BLK_E,)
    wrad = p['We1'][2 * HID].reshape(1, HID)
    wea = p['We1'][2 * HID + 1:]
    return pl.pallas_call(
        _edge_body,
        grid=grid,
        in_specs=[
            pl.BlockSpec((BLK_E, GW), lambda i: (i, 0)),
            pl.BlockSpec((BLK_E, EDGE_DIM), lambda i: (i, 0)),
            pl.BlockSpec((1, HID), lambda i: (0, 0)),
            pl.BlockSpec((EDGE_DIM, HID), lambda i: (0, 0)),
            pl.BlockSpec((1, HID), lambda i: (0, 0)),
            pl.BlockSpec((HID, HID), lambda i: (0, 0)),
            pl.BlockSpec((1, HID), lambda i: (0, 0)),
            pl.BlockSpec((HID, HID), lambda i: (0, 0)),
            pl.BlockSpec((1, HID), lambda i: (0, 0)),
            pl.BlockSpec((HID, 1), lambda i: (0, 0)),
        ],
        out_specs=pl.BlockSpec((BLK_E, TW), lambda i: (i, 0)),
        out_shape=jax.ShapeDtypeStruct((E_PAD, TW), jnp.float32),
    )(A, EA, wrad, wea, p['be1'].reshape(1, HID), p['We2'],
      p['be2'].reshape(1, HID), p['Wc1'], p['bc1'].reshape(1, HID), p['Wc2'])


def _node_body(h_ref, c_ref, agg_ref, wa_ref, wb_ref, bn1_ref, wn2_ref,
               bn2_ref, ho_ref, co_ref):
    h = h_ref[...]
    agg = agg_ref[...]
    am = agg[:, :HID]
    at = agg[:, HID:TW]
    cnt = at[:, 3:4]
    recip = 1.0 / jnp.maximum(cnt, 1.0)
    lane = lax.broadcasted_iota(jnp.int32, (1, 16), 1)
    mask3 = jnp.where(lane < 3, 1.0, 0.0).astype(jnp.float32)
    co_ref[...] = c_ref[...] + at * recip * mask3
    t = _silu(_dot(h, wa_ref[...]) + _dot(am, wb_ref[...]) + bn1_ref[...])
    ho_ref[...] = h + _dot(t, wn2_ref[...]) + bn2_ref[...]


def _node(h, coordp, AGG, p):
    grid = (N_PAD // BLK_N,)
    return pl.pallas_call(
        _node_body,
        grid=grid,
        in_specs=[
            pl.BlockSpec((BLK_N, HID), lambda i: (i, 0)),
            pl.BlockSpec((BLK_N, 16), lambda i: (i, 0)),
            pl.BlockSpec((BLK_N, TW), lambda i: (i, 0)),
            pl.BlockSpec((HID, HID), lambda i: (0, 0)),
            pl.BlockSpec((HID, HID), lambda i: (0, 0)),
            pl.BlockSpec((1, HID), lambda i: (0, 0)),
            pl.BlockSpec((HID, HID), lambda i: (0, 0)),
            pl.BlockSpec((1, HID), lambda i: (0, 0)),
        ],
        out_specs=[
            pl.BlockSpec((BLK_N, HID), lambda i: (i, 0)),
            pl.BlockSpec((BLK_N, 16), lambda i: (i, 0)),
        ],
        out_shape=[
            jax.ShapeDtypeStruct((N_PAD, HID), jnp.float32),
            jax.ShapeDtypeStruct((N_PAD, 16), jnp.float32),
        ],
    )(h, coordp, AGG, p['Wn1'][:HID], p['Wn1'][HID:],
      p['bn1'].reshape(1, HID), p['Wn2'], p['bn2'].reshape(1, HID))


def _post_body(h_ref, w_ref, b_ref, o_ref):
    o_ref[...] = _dot(h_ref[...], w_ref[...]) + b_ref[...]


def _post(h, w, b):
    grid = (N_PAD // BLK_N,)
    return pl.pallas_call(
        _post_body,
        grid=grid,
        in_specs=[
            pl.BlockSpec((BLK_N, HID), lambda i: (i, 0)),
            pl.BlockSpec((HID, HID), lambda i: (0, 0)),
            pl.BlockSpec((1, HID), lambda i: (0, 0)),
        ],
        out_specs=pl.BlockSpec((BLK_N, HID), lambda i: (i, 0)),
        out_shape=jax.ShapeDtypeStruct((N_PAD, HID), jnp.float32),
    )(h, w, b.reshape(1, HID))


def _gat_prep_body(x_ref, wl_ref, bl_ref, br_ref, att_ref, xl_ref, e_ref,
                   mx_ref):
    i = pl.program_id(0)
    xl = _dot(x_ref[...], wl_ref[...]) + bl_ref[...]
    xl_ref[...] = xl
    e = _dot(jax.nn.leaky_relu(xl + br_ref[...], 0.2), att_ref[...])
    e_ref[...] = e

    @pl.when(i == 0)
    def _init():
        mx_ref[...] = jnp.full_like(mx_ref[...], -jnp.inf)

    mx_ref[...] = jnp.maximum(mx_ref[...], jnp.max(e))


def _gat_prep(x, gp, blk):
    m = x.shape[0]
    grid = (m // blk,)
    return pl.pallas_call(
        _gat_prep_body,
        grid=grid,
        in_specs=[
            pl.BlockSpec((blk, HID), lambda i: (i, 0)),
            pl.BlockSpec((HID, HID), lambda i: (0, 0)),
            pl.BlockSpec((1, HID), lambda i: (0, 0)),
            pl.BlockSpec((1, HID), lambda i: (0, 0)),
            pl.BlockSpec((HID, 1), lambda i: (0, 0)),
        ],
        out_specs=[
            pl.BlockSpec((blk, HID), lambda i: (i, 0)),
            pl.BlockSpec((blk, 1), lambda i: (i, 0)),
            pl.BlockSpec((1, 1), lambda i: (0, 0)),
        ],
        out_shape=[
            jax.ShapeDtypeStruct((m, HID), jnp.float32),
            jax.ShapeDtypeStruct((m, 1), jnp.float32),
            jax.ShapeDtypeStruct((1, 1), jnp.float32),
        ],
    )(x, gp['Wl'], gp['bl'].reshape(1, HID), gp['br'].reshape(1, HID),
      gp['att'].reshape(HID, 1))


def _gat_w_body(xl_ref, e_ref, mx_ref, o_ref):
    w = jnp.exp(e_ref[...] - mx_ref[...])
    lane = lax.broadcasted_iota(jnp.int32, (1, 16), 1)
    one0 = jnp.where(lane == 0, 1.0, 0.0).astype(jnp.float32)
    o_ref[...] = jnp.concatenate([w * xl_ref[...], w * one0], axis=1)


def _gat_w(xl, e, mx, blk):
    m = xl.shape[0]
    grid = (m // blk,)
    return pl.pallas_call(
        _gat_w_body,
        grid=grid,
        in_specs=[
            pl.BlockSpec((blk, HID), lambda i: (i, 0)),
            pl.BlockSpec((blk, 1), lambda i: (i, 0)),
            pl.BlockSpec((1, 1), lambda i: (0, 0)),
        ],
        out_specs=pl.BlockSpec((blk, TW), lambda i: (i, 0)),
        out_shape=jax.ShapeDtypeStruct((m, TW), jnp.float32),
    )(xl, e, mx)


def _gat_out_body(agg_ref, mx_ref, c0_ref, bl_ref, bias_ref, o_ref):
    wself = jnp.exp(c0_ref[...] - mx_ref[...])
    numer = agg_ref[:, :HID] + wself * bl_ref[...]
    denom = agg_ref[:, HID:HID + 1] + wself + 1e-16
    o_ref[...] = numer / denom + bias_ref[...]


def _gat_out(agg, mx, c0, gp, blk):
    c = agg.shape[0]
    grid = (c // blk,)
    return pl.pallas_call(
        _gat_out_body,
        grid=grid,
        in_specs=[
            pl.BlockSpec((blk, TW), lambda i: (i, 0)),
            pl.BlockSpec((1, 1), lambda i: (0, 0)),
            pl.BlockSpec((1, 1), lambda i: (0, 0)),
            pl.BlockSpec((1, HID), lambda i: (0, 0)),
            pl.BlockSpec((1, HID), lambda i: (0, 0)),
        ],
        out_specs=pl.BlockSpec((blk, HID), lambda i: (i, 0)),
        out_shape=jax.ShapeDtypeStruct((c, HID), jnp.float32),
    )(agg, mx, c0, gp['bl'].reshape(1, HID), gp['bias'].reshape(1, HID))


_MESH = plsc.VectorSubcoreMesh(core_axis_name="c", subcore_axis_name="s")
NC = 2
NS = 16


def _sc_gather(table, idx2d, width):
    blks = idx2d.shape[0]
    per_w = blks // (NC * NS)

    def body(table_h, idx_h, out_h, idx_all, rows_v, g0, g1, s0, s1):
        c = lax.axis_index("c")
        s = lax.axis_index("s")
        wid = s * NC + c
        base = wid * per_w
        gsem = (g0, g1)
        ssem = (s0, s1)
        pltpu.sync_copy(idx_h.at[pl.ds(base, per_w)], idx_all)

        def gath(j, b):
            return pltpu.async_copy(table_h.at[idx_all.at[j]],
                                    rows_v.at[b], gsem[b])

        def stor(j, b):
            return pltpu.async_copy(rows_v.at[b],
                                    out_h.at[pl.ds((base + j) * 128, 128)],
                                    ssem[b])

        gath(0, 0)
        gath(1, 1)

        def group(gg, carry):
            for b in range(2):
                j = gg * 2 + b
                pltpu.make_async_copy(table_h.at[idx_all.at[j]],
                                      rows_v.at[b], gsem[b]).wait()
                stor(j, b)
                pltpu.make_async_copy(rows_v.at[b],
                                      out_h.at[pl.ds((base + j) * 128, 128)],
                                      ssem[b]).wait()

                @pl.when(j + 2 < per_w)
                def _next():
                    gath(j + 2, b)
            return carry

        lax.fori_loop(0, per_w // 2, group, 0)

    f = pl.kernel(
        body,
        out_type=jax.ShapeDtypeStruct((blks * 128, width), jnp.float32),
        mesh=_MESH,
        scratch_types=[
            pltpu.VMEM((per_w, 128), jnp.int32),
            pltpu.VMEM((2, 128, width), jnp.float32),
            pltpu.SemaphoreType.DMA,
            pltpu.SemaphoreType.DMA,
            pltpu.SemaphoreType.DMA,
            pltpu.SemaphoreType.DMA,
        ],
    )
    return f(table, idx2d)


def _sc_gather_pair(t1, t2, idx1, idx2, width):
    blks = idx1.shape[0]
    per_w = blks // (NC * NS)

    def body(t1_h, t2_h, i1_h, i2_h, out_h, i1_all, i2_all, rows_v,
             g0, g1, a0, a1, s0, s1):
        c = lax.axis_index("c")
        s = lax.axis_index("s")
        wid = s * NC + c
        base = wid * per_w
        gsem = (g0, g1)
        asem = (a0, a1)
        ssem = (s0, s1)
        pltpu.sync_copy(i1_h.at[pl.ds(base, per_w)], i1_all)
        pltpu.sync_copy(i2_h.at[pl.ds(base, per_w)], i2_all)

        def gath(j, b):
            pltpu.async_copy(t1_h.at[i1_all.at[j]], rows_v.at[b], gsem[b])

        gath(0, 0)
        gath(1, 1)

        def group(gg, carry):
            for b in range(2):
                j = gg * 2 + b
                pltpu.make_async_copy(t1_h.at[i1_all.at[j]],
                                      rows_v.at[b], gsem[b]).wait()
                pltpu.async_copy(t2_h.at[i2_all.at[j]], rows_v.at[b],
                                 asem[b], add=True).wait()
                pltpu.async_copy(rows_v.at[b],
                                 out_h.at[pl.ds((base + j) * 128, 128)],
                                 ssem[b])
                pltpu.make_async_copy(rows_v.at[b],
                                      out_h.at[pl.ds((base + j) * 128, 128)],
                                      ssem[b]).wait()

                @pl.when(j + 2 < per_w)
                def _next():
                    gath(j + 2, b)
            return carry

        lax.fori_loop(0, per_w // 2, group, 0)

    f = pl.kernel(
        body,
        out_type=jax.ShapeDtypeStruct((blks * 128, width), jnp.float32),
        mesh=_MESH,
        scratch_types=[
            pltpu.VMEM((per_w, 128), jnp.int32),
            pltpu.VMEM((per_w, 128), jnp.int32),
            pltpu.VMEM((2, 128, width), jnp.float32),
            pltpu.SemaphoreType.DMA,
            pltpu.SemaphoreType.DMA,
            pltpu.SemaphoreType.DMA,
            pltpu.SemaphoreType.DMA,
            pltpu.SemaphoreType.DMA,
            pltpu.SemaphoreType.DMA,
        ],
    )
    return f(t1, t2, idx1, idx2)


def _sc_scatter_add(S3, idx2d, zeros, acc_rows, dump_chunk):
    blks = idx2d.shape[0]
    per_tile = blks // NS
    K = 4
    outer = per_tile // K
    rows_pt = acc_rows // NS
    n_dump = rows_pt // dump_chunk

    def body(S_h, idx_h, z_h, out_h, idx_v, vals_v, dump_v, acc_sh):
        c = lax.axis_index("c")
        s = lax.axis_index("s")
        pltpu.sync_copy(z_h.at[pl.ds(s * rows_pt, rows_pt)],
                        acc_sh.at[pl.ds(s * rows_pt, rows_pt)])
        plsc.subcore_barrier()
        base = s * per_tile

        def it(g, carry):
            b0 = base + g * K
            pltpu.sync_copy(idx_h.at[pl.ds(b0, K)], idx_v)
            pltpu.sync_copy(S_h.at[c, pl.ds(b0 * 128, K * 128)], vals_v)
            for j in range(K):
                pltpu.sync_copy(vals_v.at[pl.ds(j * 128, 128)],
                                acc_sh.at[idx_v.at[j]], add=True)
            return carry

        lax.fori_loop(0, outer, it, 0)
        plsc.subcore_barrier()

        def dmp(d, carry):
            r0 = s * rows_pt + d * dump_chunk
            pltpu.sync_copy(acc_sh.at[pl.ds(r0, dump_chunk)], dump_v)
            pltpu.sync_copy(dump_v, out_h.at[c, pl.ds(r0, dump_chunk)])
            return carry

        lax.fori_loop(0, n_dump, dmp, 0)

    f = pl.kernel(
        body,
        out_type=jax.ShapeDtypeStruct((2, acc_rows, 40), jnp.float32),
        mesh=_MESH,
        scratch_types=[
            pltpu.VMEM((K, 128), jnp.int32),
            pltpu.VMEM((K * 128, 40), jnp.float32),
            pltpu.VMEM((dump_chunk, 40), jnp.float32),
            pltpu.VMEM_SHARED((acc_rows, 40), jnp.float32),
        ],
    )
    return f(S3, idx2d, zeros)


def _gather_rows(table, idx):
    return jnp.take(table, idx, axis=0)


def _scatter_add(values, ids, num_segments):
    return jax.ops.segment_sum(values, ids, num_segments=num_segments)


@jax.jit
def _run(x, pos, edge_index, edge_attr, batch_lig, batch_pro,
         lin_W, lin_b, W_in, b_in, layers, W_out, b_out, gat_lig, gat_pro):
    f32 = jnp.float32
    xp = jnp.zeros((N_PAD, NODE_DIM), f32).at[:N].set(x)
    coordp = jnp.zeros((N_PAD, 16), f32).at[:N, :3].set(pos)
    row = edge_index[0]
    col = edge_index[1]
    rowp = jnp.full((E_PAD,), N, jnp.int32).at[:E].set(row)
    colp = jnp.full((E_PAD,), N, jnp.int32).at[:E].set(col)
    row2d = rowp.reshape(E_PAD // 128, 128)
    col2d = colp.reshape(E_PAD // 128, 128)
    eap = jnp.zeros((E_PAD, EDGE_DIM), f32).at[:E].set(edge_attr)

    h = _mlp_in(xp, lin_W, lin_b, W_in, b_in)
    for p in layers:
        T1, T2 = _prep(h, coordp, p['We1'][:HID], p['We1'][HID:2 * HID])
        U = _sc_gather_pair(T1, T2, row2d, col2d, GW)
        S = _edge(U, eap, p)
        AGG = _scatter_add(S, rowp, N_PAD)
        h, coordp = _node(h, coordp, AGG, p)

    hout = _post(h, W_out, b_out)

    def gat(xa, batch, c, blk_a, blk_c, gp):
        xl, e, mx = _gat_prep(xa, gp, blk_a)
        c0 = jnp.dot(jax.nn.leaky_relu(gp['bl'] + gp['br'], 0.2),
                     gp['att']).reshape(1, 1)
        mx = jnp.maximum(mx, c0)
        S = _gat_w(xl, e, mx, blk_a)
        agg = _scatter_add(S, batch, c)
        return _gat_out(agg, mx, c0, gp, blk_c)

    x_lig_cg = gat(hout[:L], batch_lig, C_LIG, 1000, 1000, gat_lig)
    x_pro_cg = gat(hout[L:L + P], batch_pro, C_PRO, 1600, 1000, gat_pro)
    pos_lig = coordp[:L, :3]
    pos_pro = coordp[L:L + P, :3]
    return x_lig_cg, x_pro_cg, pos_lig, pos_pro


def kernel(x, pos, edge_index, edge_attr, batch_lig, batch_pro,
           lin_W, lin_b, W_in, b_in, layers, W_out, b_out, gat_lig, gat_pro):
    return _run(x, pos, edge_index, edge_attr, batch_lig, batch_pro,
                lin_W, lin_b, W_in, b_in, layers, W_out, b_out,
                gat_lig, gat_pro)
